# Initial kernel scaffold; baseline (speedup 1.0000x reference)
import jax
import jax.numpy as jnp
from jax.experimental import pallas as pl


def kernel(input_ids, embed_table, proj_w):
    raise NotImplementedError("write your pallas kernel here")


# speedup vs baseline: 1.2997x; 1.2997x over previous
import functools

import jax
import jax.numpy as jnp
from jax import lax
from jax.experimental import pallas as pl
from jax.experimental.pallas import tpu as pltpu
from jax.experimental.pallas import tpu_sc as plsc

_NUM_BUCKETS = 100000
_LANES = 16
_IDX_CHUNK = 128


def _sc_hash_gather(ids_flat, prev_flat, table):
    T = ids_flat.shape[0]
    V, D = table.shape
    info = plsc.get_sparse_core_info()
    NC, NS = info.num_cores, info.num_subcores
    NW = NC * NS
    b_per_w = T // NW
    assert T % NW == 0 and b_per_w % _IDX_CHUNK == 0

    mesh = plsc.VectorSubcoreMesh(core_axis_name="c", subcore_axis_name="s")

    @functools.partial(
        pl.kernel,
        mesh=mesh,
        out_type=jax.ShapeDtypeStruct((T, D), jnp.float32),
        scratch_types=[
            pltpu.VMEM((b_per_w,), jnp.int32),
            pltpu.VMEM((b_per_w,), jnp.int32),
            pltpu.VMEM((b_per_w, D), jnp.float32),
            pltpu.SemaphoreType.DMA,
        ],
    )
    def k(ids_hbm, prev_hbm, table_hbm, out_hbm, cur_v, idx_v, rows_v, sem):
        wid = lax.axis_index("s") * NC + lax.axis_index("c")
        base = wid * b_per_w
        pltpu.sync_copy(ids_hbm.at[pl.ds(base, b_per_w)], cur_v)
        pltpu.sync_copy(prev_hbm.at[pl.ds(base, b_per_w)], idx_v)

        def body(i, carry):
            sl = pl.ds(i * _LANES, _LANES)
            idx_v[sl] = (idx_v[sl] * 1009 + cur_v[sl]) % _NUM_BUCKETS
            return carry

        lax.fori_loop(0, b_per_w // _LANES, body, 0)

        copies = []
        for j in range(b_per_w // _IDX_CHUNK):
            sl = pl.ds(j * _IDX_CHUNK, _IDX_CHUNK)
            copies.append(
                pltpu.async_copy(table_hbm.at[idx_v.at[sl]], rows_v.at[sl], sem)
            )
        for c in copies:
            c.wait()
        pltpu.sync_copy(rows_v, out_hbm.at[pl.ds(base, b_per_w)])

    return k(ids_flat, prev_flat, table)


def _tc_project(e_flat, proj_w):
    T, D = e_flat.shape
    M = proj_w.shape[0]
    BM = 512

    def mm(x_ref, w_ref, o_ref):
        o_ref[...] = lax.dot_general(
            x_ref[...], w_ref[...], (((1,), (1,)), ((), ())),
            preferred_element_type=jnp.float32,
        )

    return pl.pallas_call(
        mm,
        grid=(T // BM,),
        in_specs=[
            pl.BlockSpec((BM, D), lambda i: (i, 0)),
            pl.BlockSpec((M, D), lambda i: (0, 0)),
        ],
        out_specs=pl.BlockSpec((BM, M), lambda i: (i, 0)),
        out_shape=jax.ShapeDtypeStruct((T, M), jnp.float32),
    )(e_flat, proj_w)


def kernel(input_ids, embed_table, proj_w):
    B, S = input_ids.shape
    M = proj_w.shape[0]
    ids = input_ids.astype(jnp.int32)
    prev = jnp.pad(ids[:, :-1], ((0, 0), (1, 0)))
    e = _sc_hash_gather(ids.reshape(-1), prev.reshape(-1), embed_table)
    out = _tc_project(e, proj_w)
    return out.reshape(B, S, M)
